# Initial kernel scaffold; baseline (speedup 1.0000x reference)
import jax
import jax.numpy as jnp
from jax.experimental import pallas as pl


def kernel(c, v, edge_index, e, gC_W1, gC_b1, gC_W2, gC_b2, gV_W1, gV_b1, gV_W2, gV_b2, fC_W1, fC_b1, fC_W2, fC_b2, fV_W1, fV_b1, fV_W2, fV_b2, beta_c, sigma_c, beta_v, sigma_v):
    raise NotImplementedError("write your pallas kernel here")


# speedup vs baseline: 1.6804x; 1.6804x over previous
import functools

import jax
import jax.numpy as jnp
from jax import lax
from jax.experimental import pallas as pl
from jax.experimental.pallas import tpu as pltpu
from jax.experimental.pallas import tpu_sc as plsc

F32 = jnp.float32

_NC = 2
_NS = 16
_NW = _NC * _NS
_CHUNK = 128


def _mm_body(x_ref, w_ref, o_ref):
    o_ref[...] = jnp.dot(x_ref[...], w_ref[...], preferred_element_type=F32)


def _tc_matmul(x, w, block_m):
    m, k = x.shape
    _, n = w.shape
    grid = m // block_m
    return pl.pallas_call(
        _mm_body,
        grid=(grid,),
        in_specs=[
            pl.BlockSpec((block_m, k), lambda i: (i, 0)),
            pl.BlockSpec((k, n), lambda i: (0, 0)),
        ],
        out_specs=pl.BlockSpec((block_m, n), lambda i: (i, 0)),
        out_shape=jax.ShapeDtypeStruct((m, n), F32),
    )(x, w)


def _edge_mlp_body(g_ref, e_ref, w1e_ref, b1_ref, w2_ref, b2_ref, o_ref):
    h = (g_ref[0, ...] + g_ref[1, ...]
         + jnp.dot(e_ref[...], w1e_ref[...], preferred_element_type=F32)
         + b1_ref[...])
    h = jnp.maximum(h, 0.0)
    msg = jnp.dot(h, w2_ref[...], preferred_element_type=F32) + b2_ref[...]
    msg = jnp.maximum(msg, 0.0)
    o_ref[0, ...] = msg[:, :128]
    o_ref[1, ...] = msg[:, 128:]


def _tc_edge_mlp(g, e, w1e, b1, w2, b2, block_e):
    _, n_edges, emb = g.shape
    edim = e.shape[1]
    grid = n_edges // block_e
    return pl.pallas_call(
        _edge_mlp_body,
        grid=(grid,),
        in_specs=[
            pl.BlockSpec((2, block_e, emb), lambda i: (0, i, 0)),
            pl.BlockSpec((block_e, edim), lambda i: (i, 0)),
            pl.BlockSpec((edim, emb), lambda i: (0, 0)),
            pl.BlockSpec((1, emb), lambda i: (0, 0)),
            pl.BlockSpec((emb, emb), lambda i: (0, 0)),
            pl.BlockSpec((1, emb), lambda i: (0, 0)),
        ],
        out_specs=pl.BlockSpec((2, block_e, 128), lambda i: (0, i, 0)),
        out_shape=jax.ShapeDtypeStruct((2, n_edges, 128), F32),
    )(g, e, w1e, b1, w2, b2)


def _node_mlp_body(x_ref, a_ref, w1x_ref, w1a_ref, b1_ref, w2_ref, b2_ref,
                   wp_ref, o_ref, p_ref):
    agg = jnp.concatenate([a_ref[0, ...], a_ref[1, ...]], axis=-1)
    t = (jnp.dot(x_ref[...], w1x_ref[...], preferred_element_type=F32)
         + jnp.dot(agg, w1a_ref[...], preferred_element_type=F32)
         + b1_ref[...])
    t = jnp.maximum(t, 0.0)
    y = jnp.dot(t, w2_ref[...], preferred_element_type=F32) + b2_ref[...]
    y = jnp.maximum(y, 0.0)
    o_ref[...] = y
    p_ref[...] = jnp.dot(y, wp_ref[...], preferred_element_type=F32)


def _tc_node_mlp(x, agg2, w1x, w1a, b1, w2, b2, wp, block_m):
    n, emb = x.shape
    grid = n // block_m
    return pl.pallas_call(
        _node_mlp_body,
        grid=(grid,),
        in_specs=[
            pl.BlockSpec((block_m, emb), lambda i: (i, 0)),
            pl.BlockSpec((2, block_m, 128), lambda i: (0, i, 0)),
            pl.BlockSpec((emb, emb), lambda i: (0, 0)),
            pl.BlockSpec((emb, emb), lambda i: (0, 0)),
            pl.BlockSpec((1, emb), lambda i: (0, 0)),
            pl.BlockSpec((emb, emb), lambda i: (0, 0)),
            pl.BlockSpec((1, emb), lambda i: (0, 0)),
            pl.BlockSpec((emb, emb), lambda i: (0, 0)),
        ],
        out_specs=[
            pl.BlockSpec((block_m, emb), lambda i: (i, 0)),
            pl.BlockSpec((block_m, emb), lambda i: (i, 0)),
        ],
        out_shape=[
            jax.ShapeDtypeStruct((n, emb), F32),
            jax.ShapeDtypeStruct((n, emb), F32),
        ],
    )(x, agg2, w1x, w1a, b1, w2, b2, wp)


def _sc_gather_body(n_edges, pc_hbm, pv_hbm, row_hbm, col_hbm, g_hbm,
                    idxr_v, idxc_v, gbuf_v, tbuf_v, sem1, sem2):
    cid = lax.axis_index("c")
    sid = lax.axis_index("s")
    w = sid * _NC + cid
    chunks = n_edges // _CHUNK
    n_full = chunks // _NW
    n_rem = chunks - n_full * _NW
    nloc = n_full + jnp.where(w < n_rem, 1, 0)

    def body(k, _):
        base = (w + _NW * k) * _CHUNK
        pltpu.sync_copy(row_hbm.at[pl.ds(base, _CHUNK)], idxr_v)
        pltpu.sync_copy(col_hbm.at[pl.ds(base, _CHUNK)], idxc_v)
        cp1 = pltpu.async_copy(pc_hbm.at[idxr_v], gbuf_v, sem1)
        cp2 = pltpu.async_copy(pv_hbm.at[idxc_v], tbuf_v, sem2)
        cp1.wait()
        cp2.wait()
        pltpu.sync_copy(gbuf_v, g_hbm.at[0, pl.ds(base, _CHUNK)])
        pltpu.sync_copy(tbuf_v, g_hbm.at[1, pl.ds(base, _CHUNK)])
        return 0

    lax.fori_loop(0, nloc, body, 0)


def _sc_gather_combine(pc, pv, row, col):
    n_edges = row.shape[0]
    emb = pc.shape[1]
    mesh = plsc.VectorSubcoreMesh(core_axis_name="c", subcore_axis_name="s")
    return pl.kernel(
        functools.partial(_sc_gather_body, n_edges),
        out_type=jax.ShapeDtypeStruct((2, n_edges, emb), F32),
        mesh=mesh,
        scratch_types=[
            pltpu.VMEM((_CHUNK,), jnp.int32),
            pltpu.VMEM((_CHUNK,), jnp.int32),
            pltpu.VMEM((_CHUNK, emb), F32),
            pltpu.VMEM((_CHUNK, emb), F32),
            pltpu.SemaphoreType.DMA,
            pltpu.SemaphoreType.DMA,
        ],
    )(pc, pv, row, col)


def _sc_scatter_body(n_edges, rows_per_tile, msg_hbm, idx_hbm, zero_hbm,
                     agg_hbm, idx_v, msgb_v, zbuf_v, acc_sh):
    cid = lax.axis_index("c")
    sid = lax.axis_index("s")
    rbase = sid * rows_per_tile

    n_zc = rows_per_tile // _CHUNK
    pltpu.sync_copy(zero_hbm, zbuf_v)
    for z in range(n_zc):
        pltpu.sync_copy(zbuf_v, acc_sh.at[pl.ds(rbase + z * _CHUNK, _CHUNK)])
    plsc.subcore_barrier()

    chunks = n_edges // _CHUNK
    n_full = chunks // _NS
    n_rem = chunks - n_full * _NS
    nloc = n_full + jnp.where(sid < n_rem, 1, 0)

    def body(k, _):
        base = (sid + _NS * k) * _CHUNK
        pltpu.sync_copy(idx_hbm.at[pl.ds(base, _CHUNK)], idx_v)
        pltpu.sync_copy(msg_hbm.at[cid, pl.ds(base, _CHUNK)], msgb_v)
        pltpu.sync_copy(msgb_v, acc_sh.at[idx_v], add=True)
        return 0

    lax.fori_loop(0, nloc, body, 0)
    plsc.subcore_barrier()

    for z in range(n_zc):
        pltpu.sync_copy(acc_sh.at[pl.ds(rbase + z * _CHUNK, _CHUNK)], zbuf_v)
        pltpu.sync_copy(zbuf_v, agg_hbm.at[cid, pl.ds(rbase + z * _CHUNK, _CHUNK)])


def _sc_scatter_add(msg2, idx, n_nodes):
    n_edges = idx.shape[0]
    half = msg2.shape[2]
    rows_per_tile = (-(-n_nodes // _NS) + _CHUNK - 1) // _CHUNK * _CHUNK
    n_pad = rows_per_tile * _NS
    zero = jnp.zeros((_CHUNK, half), F32)
    mesh = plsc.VectorSubcoreMesh(core_axis_name="c", subcore_axis_name="s")
    return pl.kernel(
        functools.partial(_sc_scatter_body, n_edges, rows_per_tile),
        out_type=jax.ShapeDtypeStruct((2, n_pad, half), F32),
        mesh=mesh,
        scratch_types=[
            pltpu.VMEM((_CHUNK,), jnp.int32),
            pltpu.VMEM((_CHUNK, half), F32),
            pltpu.VMEM((_CHUNK, half), F32),
            pltpu.VMEM_SHARED((n_pad, half), F32),
        ],
    )(msg2, idx, zero)


def kernel(c, v, edge_index, e,
           gC_W1, gC_b1, gC_W2, gC_b2,
           gV_W1, gV_b1, gV_W2, gV_b2,
           fC_W1, fC_b1, fC_W2, fC_b2,
           fV_W1, fV_b1, fV_W2, fV_b2,
           beta_c, sigma_c, beta_v, sigma_v):
    n_c, emb = c.shape
    n_v = v.shape[0]
    row = edge_index[0]
    col = edge_index[1]

    gC_W1c, gC_W1v, gC_W1e = gC_W1[:emb], gC_W1[emb:2 * emb], gC_W1[2 * emb:]
    gV_W1c, gV_W1v, gV_W1e = gV_W1[:emb], gV_W1[emb:2 * emb], gV_W1[2 * emb:]

    fC_W1x, fC_W1a = fC_W1[:emb], fC_W1[emb:]
    fV_W1x, fV_W1a = fV_W1[:emb], fV_W1[emb:]
    fC_W1a_eff = fC_W1a / sigma_c[:, None]
    fC_b1_eff = fC_b1 - (beta_c / sigma_c) @ fC_W1a
    fV_W1a_eff = fV_W1a / sigma_v[:, None]
    fV_b1_eff = fV_b1 - (beta_v / sigma_v) @ fV_W1a

    b2 = lambda x: x.reshape(1, -1)

    pc1 = _tc_matmul(c, gC_W1c, 1000)
    pv_both = _tc_matmul(v, jnp.concatenate([gC_W1v, gV_W1v], axis=1), 1000)
    pv1 = pv_both[:, :emb]
    pv2 = pv_both[:, emb:]

    g1 = _sc_gather_combine(pc1, pv1, row, col)
    msg1 = _tc_edge_mlp(g1, e, gC_W1e, b2(gC_b1), gC_W2, b2(gC_b2), 640)
    agg_c = _sc_scatter_add(msg1, row, n_c)
    c_new, pc2 = _tc_node_mlp(c, agg_c, fC_W1x, fC_W1a_eff, b2(fC_b1_eff),
                              fC_W2, b2(fC_b2), gV_W1c, 1000)

    g2 = _sc_gather_combine(pc2, pv2, row, col)
    msg2 = _tc_edge_mlp(g2, e, gV_W1e, b2(gV_b1), gV_W2, b2(gV_b2), 640)
    agg_v = _sc_scatter_add(msg2, col, n_v)
    v_new, _ = _tc_node_mlp(v, agg_v, fV_W1x, fV_W1a_eff, b2(fV_b1_eff),
                            fV_W2, b2(fV_b2), fV_W2, 1000)

    return (c_new, v_new)
